# baseline (speedup 1.0000x reference)
import functools

import jax
import jax.numpy as jnp
from jax.experimental import pallas as pl
from jax.experimental.pallas import tpu as pltpu

D_MODEL = 1024
N_EXP = 209
E_PAD = 256
T_BLK = 1024


def _gate_kernel(x_ref, w_ref, xout_ref, idx_ref, score_ref, counts_ref,
                 psum_ref, loss_ref, copy_sem, *, n_tokens, n_blocks):
    step = pl.program_id(0)
    copy = pltpu.make_async_copy(
        x_ref, xout_ref.at[pl.ds(step * T_BLK, T_BLK), :], copy_sem)
    copy.start()

    @pl.when(step == 0)
    def _init():
        counts_ref[...] = jnp.zeros_like(counts_ref)
        psum_ref[...] = jnp.zeros_like(psum_ref)

    logits_t = jax.lax.dot_general(
        w_ref[...], x_ref[...],
        dimension_numbers=(((1,), (1,)), ((), ())),
        preferred_element_type=jnp.float32)

    e_col = jax.lax.broadcasted_iota(jnp.int32, (E_PAD, 1), 0)
    ex = jnp.exp(jnp.where(e_col < N_EXP, logits_t, -1e30))

    m_row = jnp.max(ex, axis=0, keepdims=True)
    ones_e = jnp.ones((1, E_PAD), jnp.float32)
    denom = jax.lax.dot_general(
        ones_e, ex, dimension_numbers=(((1,), (0,)), ((), ())),
        preferred_element_type=jnp.float32)
    score = m_row / denom
    r = 1.0 / denom

    oh = jnp.where(ex == m_row, 1.0, 0.0)
    iota_e = jax.lax.broadcasted_iota(jnp.int32, (1, E_PAD), 1).astype(jnp.float32)
    idx_f = jax.lax.dot_general(
        iota_e, oh, dimension_numbers=(((1,), (0,)), ((), ())),
        preferred_element_type=jnp.float32)

    idx_ref[...] = idx_f.astype(jnp.int32).reshape(1, 1, T_BLK)
    score_ref[...] = score.reshape(1, 1, T_BLK)

    ones_t = jnp.ones((1, T_BLK), jnp.float32)
    counts_ref[...] += jax.lax.dot_general(
        ones_t, oh, dimension_numbers=(((1,), (1,)), ((), ())),
        preferred_element_type=jnp.float32)
    psum_ref[...] += jax.lax.dot_general(
        r, ex, dimension_numbers=(((1,), (1,)), ((), ())),
        preferred_element_type=jnp.float32)

    @pl.when(step == n_blocks - 1)
    def _fin():
        c = counts_ref[...]
        p = psum_ref[...]
        loss = (N_EXP / (n_tokens * n_tokens)) * jnp.sum(p * c)
        loss_ref[...] = jnp.full((1, 128), loss, jnp.float32)
        counts_ref[...] = 0.1 * c
        psum_ref[...] = 0.1 * p

    copy.wait()


def kernel(x, gate_weight):
    batch_size, seq_len, d_model = x.shape
    x_flat = x.reshape(-1, d_model)
    n_tokens = x_flat.shape[0]
    n_blocks = n_tokens // T_BLK

    body = functools.partial(_gate_kernel, n_tokens=n_tokens, n_blocks=n_blocks)
    x_out, idx3, score3, counts, psum, loss_v = pl.pallas_call(
        body,
        grid=(n_blocks,),
        in_specs=[
            pl.BlockSpec((T_BLK, D_MODEL), lambda i: (i, 0)),
            pl.BlockSpec((E_PAD, D_MODEL), lambda i: (0, 0)),
        ],
        out_specs=[
            pl.BlockSpec(memory_space=pl.ANY),
            pl.BlockSpec((1, 1, T_BLK), lambda i: (i, 0, 0)),
            pl.BlockSpec((1, 1, T_BLK), lambda i: (i, 0, 0)),
            pl.BlockSpec((1, E_PAD), lambda i: (0, 0)),
            pl.BlockSpec((1, E_PAD), lambda i: (0, 0)),
            pl.BlockSpec((1, 128), lambda i: (0, 0)),
        ],
        out_shape=[
            jax.ShapeDtypeStruct((n_tokens, d_model), jnp.float32),
            jax.ShapeDtypeStruct((n_blocks, 1, T_BLK), jnp.int32),
            jax.ShapeDtypeStruct((n_blocks, 1, T_BLK), jnp.float32),
            jax.ShapeDtypeStruct((1, E_PAD), jnp.float32),
            jax.ShapeDtypeStruct((1, E_PAD), jnp.float32),
            jax.ShapeDtypeStruct((1, 128), jnp.float32),
        ],
        scratch_shapes=[pltpu.SemaphoreType.DMA],
    )(x_flat, gate_weight)

    expert_indices = idx3.reshape(n_tokens)
    gate_scores = score3.reshape(n_tokens)
    load_balancing_loss = loss_v[0, 0]
    expert_counts = counts[0, :N_EXP]
    gate_probs_sum = psum[0, :N_EXP]
    return (x_out, expert_indices, gate_scores, load_balancing_loss,
            expert_counts, gate_probs_sum)


# speedup vs baseline: 1.1599x; 1.1599x over previous
import functools

import jax
import jax.numpy as jnp
from jax.experimental import pallas as pl
from jax.experimental.pallas import tpu as pltpu

D_MODEL = 1024
N_EXP = 209
E_PAD = 256
T_BLK = 4096


def _gate_kernel(x_ref, w_ref, xout_ref, idx_ref, score_ref, counts_ref,
                 psum_ref, loss_ref, copy_sem, *, n_tokens, n_blocks):
    step = pl.program_id(0)
    copy = pltpu.make_async_copy(
        x_ref, xout_ref.at[pl.ds(step * T_BLK, T_BLK), :], copy_sem)
    copy.start()

    @pl.when(step == 0)
    def _init():
        counts_ref[...] = jnp.zeros_like(counts_ref)
        psum_ref[...] = jnp.zeros_like(psum_ref)

    logits_t = jax.lax.dot_general(
        w_ref[...], x_ref[...],
        dimension_numbers=(((1,), (1,)), ((), ())),
        preferred_element_type=jnp.float32)

    e_col = jax.lax.broadcasted_iota(jnp.int32, (E_PAD, 1), 0)
    ex = jnp.exp(jnp.where(e_col < N_EXP, logits_t, -1e30))

    m_row = jnp.max(ex, axis=0, keepdims=True)
    ones_e = jnp.ones((1, E_PAD), jnp.float32)
    denom = jax.lax.dot_general(
        ones_e, ex, dimension_numbers=(((1,), (0,)), ((), ())),
        preferred_element_type=jnp.float32)
    score = m_row / denom
    r = 1.0 / denom

    oh = jnp.where(ex == m_row, 1.0, 0.0)
    iota_e = jax.lax.broadcasted_iota(jnp.int32, (1, E_PAD), 1).astype(jnp.float32)
    idx_f = jax.lax.dot_general(
        iota_e, oh, dimension_numbers=(((1,), (0,)), ((), ())),
        preferred_element_type=jnp.float32)

    idx_ref[...] = idx_f.astype(jnp.int32).reshape(1, 1, T_BLK)
    score_ref[...] = score.reshape(1, 1, T_BLK)

    ones_t = jnp.ones((1, T_BLK), jnp.float32)
    counts_ref[...] += jax.lax.dot_general(
        ones_t, oh, dimension_numbers=(((1,), (1,)), ((), ())),
        preferred_element_type=jnp.float32)
    psum_ref[...] += jax.lax.dot_general(
        r, ex, dimension_numbers=(((1,), (1,)), ((), ())),
        preferred_element_type=jnp.float32)

    @pl.when(step == n_blocks - 1)
    def _fin():
        c = counts_ref[...]
        p = psum_ref[...]
        loss = (N_EXP / (n_tokens * n_tokens)) * jnp.sum(p * c)
        loss_ref[...] = jnp.full((1, 128), loss, jnp.float32)
        counts_ref[...] = 0.1 * c
        psum_ref[...] = 0.1 * p

    copy.wait()


def kernel(x, gate_weight):
    batch_size, seq_len, d_model = x.shape
    x_flat = x.reshape(-1, d_model)
    n_tokens = x_flat.shape[0]
    n_blocks = n_tokens // T_BLK

    body = functools.partial(_gate_kernel, n_tokens=n_tokens, n_blocks=n_blocks)
    x_out, idx3, score3, counts, psum, loss_v = pl.pallas_call(
        body,
        grid=(n_blocks,),
        in_specs=[
            pl.BlockSpec((T_BLK, D_MODEL), lambda i: (i, 0)),
            pl.BlockSpec((E_PAD, D_MODEL), lambda i: (0, 0)),
        ],
        out_specs=[
            pl.BlockSpec(memory_space=pl.ANY),
            pl.BlockSpec((1, 1, T_BLK), lambda i: (i, 0, 0)),
            pl.BlockSpec((1, 1, T_BLK), lambda i: (i, 0, 0)),
            pl.BlockSpec((1, E_PAD), lambda i: (0, 0)),
            pl.BlockSpec((1, E_PAD), lambda i: (0, 0)),
            pl.BlockSpec((1, 128), lambda i: (0, 0)),
        ],
        out_shape=[
            jax.ShapeDtypeStruct((n_tokens, d_model), jnp.float32),
            jax.ShapeDtypeStruct((n_blocks, 1, T_BLK), jnp.int32),
            jax.ShapeDtypeStruct((n_blocks, 1, T_BLK), jnp.float32),
            jax.ShapeDtypeStruct((1, E_PAD), jnp.float32),
            jax.ShapeDtypeStruct((1, E_PAD), jnp.float32),
            jax.ShapeDtypeStruct((1, 128), jnp.float32),
        ],
        scratch_shapes=[pltpu.SemaphoreType.DMA],
    )(x_flat, gate_weight)

    expert_indices = idx3.reshape(n_tokens)
    gate_scores = score3.reshape(n_tokens)
    load_balancing_loss = loss_v[0, 0]
    expert_counts = counts[0, :N_EXP]
    gate_probs_sum = psum[0, :N_EXP]
    return (x_out, expert_indices, gate_scores, load_balancing_loss,
            expert_counts, gate_probs_sum)
